# baseline (speedup 1.0000x reference)
import functools

import jax
import jax.numpy as jnp
from jax import lax
from jax.experimental import pallas as pl
from jax.experimental.pallas import tpu as pltpu
from jax.experimental.pallas import tpu_sc as plsc

N_NODES = 10000
FDIM = 128
HALF = FDIM // 2
NC = 2
NS = 16
CH = 128
CNTW = 16
NBUF = 4
NPH = 2

NACC = 10240
RPT = NACC // NS
DUMMY_ROW = N_NODES


def _spmm_body(with_count, cpt, table, gidx, sidx, *refs):
  if with_count:
    (acc_out, cnt_out, gi_all, si_all, rows, ones_v,
     gsem, ssem, csem, acc_sh, cnt_sh) = refs
  else:
    acc_out, gi_all, si_all, rows, gsem, ssem, acc_sh = refs

  c = lax.axis_index("c")
  s = lax.axis_index("s")
  on_sc0 = c == 0
  zero16 = jnp.zeros((16,), jnp.float32)
  one16 = jnp.ones((16,), jnp.float32)
  hcpt = cpt // NPH
  ngroups = hcpt // NBUF

  cvec = jnp.zeros((16,), jnp.int32) + c

  def zrow(i, _):
    for k in range(HALF // 16):
      rows[0, i, pl.ds(k * 16, 16)] = zero16
    return 0
  lax.fori_loop(0, CH, zrow, 0)
  r0 = pl.multiple_of(s * RPT, CH)
  for k in range(RPT // CH):
    pltpu.sync_copy(rows.at[0], acc_sh.at[pl.ds(r0 + k * CH, CH)])

  if with_count:
    @pl.when(on_sc0)
    def _():
      def zc(i, _):
        ones_v[i, :] = zero16
        return 0
      lax.fori_loop(0, CH, zc, 0)
      for k in range(RPT // CH):
        pltpu.sync_copy(ones_v.at[pl.ds(0, CH)],
                        cnt_sh.at[pl.ds(r0 + k * CH, CH)])

      def fo(i, _):
        ones_v[i, :] = one16
        return 0
      lax.fori_loop(0, CH, fo, 0)
  plsc.subcore_barrier()

  def gstart(chunk, b):
    pltpu.async_copy(table.at[gi_all.at[chunk]], rows.at[b], gsem.at[b])

  def gwait(b):
    pltpu.make_async_copy(table.at[gi_all.at[0]], rows.at[b], gsem.at[b]).wait()

  def sstart(chunk, b):
    pltpu.async_copy(rows.at[b], acc_sh.at[si_all.at[chunk]], ssem.at[b],
                     add=True)

  def swait(b):
    pltpu.make_async_copy(rows.at[b], acc_sh.at[si_all.at[0]],
                          ssem.at[b]).wait()

  for ph in range(NPH):
    pltpu.sync_copy(gidx.at[s, pl.ds(ph * hcpt, hcpt)], gi_all)
    pltpu.sync_copy(sidx.at[s, pl.ds(ph * hcpt, hcpt)], si_all)

    def xform(j, _):
      for k in range(CH // 16):
        t = gi_all[j, pl.ds(k * 16, 16)]
        gi_all[j, pl.ds(k * 16, 16)] = t + t + cvec
      return 0
    lax.fori_loop(0, hcpt, xform, 0)

    for b in range(NBUF):
      gstart(b, b)

    def group(i, _):
      i0 = i * NBUF
      for b in range(NBUF):
        gwait(b)
        sstart(i0 + b, b)
        if with_count:
          @pl.when(on_sc0)
          def _():
            pltpu.async_copy(ones_v, cnt_sh.at[si_all.at[i0 + b]], csem,
                             add=True)
      for b in range(NBUF):
        swait(b)

        @pl.when(i < ngroups - 1)
        def _():
          gstart(i0 + NBUF + b, b)
      return 0
    lax.fori_loop(0, ngroups, group, 0)

    if with_count:
      @pl.when(on_sc0)
      def _():
        def drain(j, _):
          pltpu.make_async_copy(ones_v, cnt_sh.at[si_all.at[0]], csem).wait()
          return 0
        lax.fori_loop(0, hcpt, drain, 0)
  plsc.subcore_barrier()

  pltpu.sync_copy(acc_sh.at[pl.ds(r0, RPT)], acc_out.at[c, pl.ds(r0, RPT)])
  if with_count:
    @pl.when(on_sc0)
    def _():
      pltpu.sync_copy(cnt_sh.at[pl.ds(r0, RPT)], cnt_out.at[pl.ds(r0, RPT)])


@functools.cache
def _make_spmm(with_count, cpt):
  out_acc = jax.ShapeDtypeStruct((NC, NACC, HALF), jnp.float32)
  out_cnt = jax.ShapeDtypeStruct((NACC, CNTW), jnp.float32)
  hcpt = cpt // NPH
  scratch = [
      pltpu.VMEM((hcpt, CH), jnp.int32),
      pltpu.VMEM((hcpt, CH), jnp.int32),
      pltpu.VMEM((NBUF, CH, HALF), jnp.float32),
  ]
  if with_count:
    scratch += [pltpu.VMEM((CH, CNTW), jnp.float32)]
  scratch += [
      pltpu.SemaphoreType.DMA((NBUF,)),
      pltpu.SemaphoreType.DMA((NBUF,)),
  ]
  if with_count:
    scratch += [pltpu.SemaphoreType.DMA]
  scratch += [pltpu.VMEM_SHARED((NACC, HALF), jnp.float32)]
  if with_count:
    scratch += [pltpu.VMEM_SHARED((NACC, CNTW), jnp.float32)]
  return pl.kernel(
      functools.partial(_spmm_body, with_count, cpt),
      out_type=(out_acc, out_cnt) if with_count else out_acc,
      mesh=plsc.VectorSubcoreMesh(core_axis_name="c", subcore_axis_name="s"),
      scratch_types=scratch,
      compiler_params=pltpu.CompilerParams(use_tc_tiling_on_sc=False),
  )


def _dense_body(acc_ref, cnt_ref, x_ref, wl_ref, bl_ref, wr_ref, o_ref):
  cnt = cnt_ref[:, 0]
  inv = 1.0 / jnp.maximum(cnt, 1.0)
  m0 = acc_ref[0] * inv[:, None]
  m1 = acc_ref[1] * inv[:, None]
  y = jnp.dot(m0, wl_ref[pl.ds(0, HALF), :], preferred_element_type=jnp.float32)
  y = y + jnp.dot(m1, wl_ref[pl.ds(HALF, HALF), :],
                  preferred_element_type=jnp.float32)
  y = y + bl_ref[...]
  y = y + jnp.dot(x_ref[...], wr_ref[...], preferred_element_type=jnp.float32)
  o_ref[...] = jnp.maximum(y, 0.0)


_DR = 1000


def _dense(acc, cnt, x, wl, bl, wr):
  n = x.shape[0]
  grid = n // _DR
  return pl.pallas_call(
      _dense_body,
      grid=(grid,),
      in_specs=[
          pl.BlockSpec((NC, _DR, HALF), lambda i: (0, i, 0)),
          pl.BlockSpec((_DR, CNTW), lambda i: (i, 0)),
          pl.BlockSpec((_DR, FDIM), lambda i: (i, 0)),
          pl.BlockSpec((FDIM, FDIM), lambda i: (0, 0)),
          pl.BlockSpec((1, FDIM), lambda i: (0, 0)),
          pl.BlockSpec((FDIM, FDIM), lambda i: (0, 0)),
      ],
      out_specs=pl.BlockSpec((_DR, FDIM), lambda i: (i, 0)),
      out_shape=jax.ShapeDtypeStruct((n, FDIM), jnp.float32),
  )(acc, cnt, x, wl, bl, wr)


def kernel(x_human, x_bacterial, edge_index,
           h1_Wl, h1_bl, h1_Wr, h2_Wl, h2_bl, h2_Wr,
           b1_Wl, b1_bl, b1_Wr, b2_Wl, b2_bl, b2_Wr):
  src = edge_index[0]
  dst = edge_index[1]
  e = src.shape[0]
  cpt = -(-e // (NS * CH * NBUF * NPH)) * NBUF * NPH
  ep = NS * cpt * CH
  pad = ep - e
  gpad = jnp.zeros((pad,), jnp.int32)
  spad = jnp.full((pad,), DUMMY_ROW, jnp.int32)
  g_h = jnp.concatenate([dst, gpad]).reshape(NS, cpt, CH)
  s_h = jnp.concatenate([src, spad]).reshape(NS, cpt, CH)
  g_b = jnp.concatenate([src, gpad]).reshape(NS, cpt, CH)
  s_b = jnp.concatenate([dst, spad]).reshape(NS, cpt, CH)

  spmm_count = _make_spmm(True, cpt)
  spmm = _make_spmm(False, cpt)

  def half_view(x):
    return x.reshape(-1, HALF)

  acc_h1, cnt_h = spmm_count(half_view(x_human), g_h, s_h)
  acc_b1, cnt_b = spmm_count(half_view(x_bacterial), g_b, s_b)

  cnt_h = cnt_h[:N_NODES]
  cnt_b = cnt_b[:N_NODES]

  h1 = _dense(acc_h1, cnt_h, x_human, h1_Wl, h1_bl.reshape(1, -1), h1_Wr)
  b1 = _dense(acc_b1, cnt_b, x_bacterial, b1_Wl, b1_bl.reshape(1, -1), b1_Wr)

  acc_h2 = spmm(half_view(h1), g_h, s_h)
  acc_b2 = spmm(half_view(b1), g_b, s_b)

  h2 = _dense(acc_h2, cnt_h, h1, h2_Wl, h2_bl.reshape(1, -1), h2_Wr)
  b2 = _dense(acc_b2, cnt_b, b1, b2_Wl, b2_bl.reshape(1, -1), b2_Wr)
  return (h2, b2)


# speedup vs baseline: 1.8801x; 1.8801x over previous
import functools

import jax
import jax.numpy as jnp
from jax import lax
from jax.experimental import pallas as pl
from jax.experimental.pallas import tpu as pltpu
from jax.experimental.pallas import tpu_sc as plsc

N_NODES = 10000
FDIM = 128
QW = 32
NQ = FDIM // QW
NC = 2
NS = 16
CH = 128
CNTW = 16
NBUF = 4
NPH = 2

NACC = 10240
RPT = NACC // NS
NTT = N_NODES // NS
DUMMY_ROW = N_NODES


def _spmm_body(with_count, cpt, table, gidx, sidx, *refs):
  if with_count:
    (acc_out, cnt_out, gi_all, si_all, rows, ones_v,
     gsem, ssem, csem, tab_sh, acc_sh, cnt_sh) = refs
  else:
    acc_out, gi_all, si_all, rows, gsem, ssem, tab_sh, acc_sh = refs

  c = lax.axis_index("c")
  s = lax.axis_index("s")
  on_sc0 = c == 0
  zero16 = jnp.zeros((16,), jnp.float32)
  one16 = jnp.ones((16,), jnp.float32)
  hcpt = cpt // NPH
  ngroups = hcpt // NBUF

  t0 = s * NTT
  pltpu.sync_copy(table.at[c, pl.ds(t0, NTT)], tab_sh.at[pl.ds(t0, NTT)])

  def zrow(i, _):
    for k in range(QW // 16):
      rows[0, i, pl.ds(k * 16, 16)] = zero16
    return 0
  lax.fori_loop(0, CH, zrow, 0)
  r0 = pl.multiple_of(s * RPT, CH)
  for k in range(RPT // CH):
    pltpu.sync_copy(rows.at[0], acc_sh.at[pl.ds(r0 + k * CH, CH)])

  if with_count:
    @pl.when(on_sc0)
    def _():
      def zc(i, _):
        ones_v[i, :] = zero16
        return 0
      lax.fori_loop(0, CH, zc, 0)
      for k in range(RPT // CH):
        pltpu.sync_copy(ones_v.at[pl.ds(0, CH)],
                        cnt_sh.at[pl.ds(r0 + k * CH, CH)])

      def fo(i, _):
        ones_v[i, :] = one16
        return 0
      lax.fori_loop(0, CH, fo, 0)
  plsc.subcore_barrier()

  def gstart(chunk, b):
    pltpu.async_copy(tab_sh.at[gi_all.at[chunk]], rows.at[b], gsem.at[b])

  def gwait(b):
    pltpu.make_async_copy(tab_sh.at[gi_all.at[0]], rows.at[b],
                          gsem.at[b]).wait()

  def sstart(chunk, b):
    pltpu.async_copy(rows.at[b], acc_sh.at[si_all.at[chunk]], ssem.at[b],
                     add=True)

  def swait(b):
    pltpu.make_async_copy(rows.at[b], acc_sh.at[si_all.at[0]],
                          ssem.at[b]).wait()

  for ph in range(NPH):
    pltpu.sync_copy(gidx.at[s, pl.ds(ph * hcpt, hcpt)], gi_all)
    pltpu.sync_copy(sidx.at[s, pl.ds(ph * hcpt, hcpt)], si_all)

    for b in range(NBUF):
      gstart(b, b)

    def group(i, _):
      i0 = i * NBUF
      for b in range(NBUF):
        gwait(b)
        sstart(i0 + b, b)
        if with_count:
          @pl.when(on_sc0)
          def _():
            pltpu.async_copy(ones_v, cnt_sh.at[si_all.at[i0 + b]], csem,
                             add=True)
      for b in range(NBUF):
        swait(b)

        @pl.when(i < ngroups - 1)
        def _():
          gstart(i0 + NBUF + b, b)
      return 0
    lax.fori_loop(0, ngroups, group, 0)

    if with_count:
      @pl.when(on_sc0)
      def _():
        def drain(j, _):
          pltpu.make_async_copy(ones_v, cnt_sh.at[si_all.at[0]], csem).wait()
          return 0
        lax.fori_loop(0, hcpt, drain, 0)
  plsc.subcore_barrier()

  pltpu.sync_copy(acc_sh.at[pl.ds(r0, RPT)], acc_out.at[c, pl.ds(r0, RPT)])
  if with_count:
    @pl.when(on_sc0)
    def _():
      pltpu.sync_copy(cnt_sh.at[pl.ds(r0, RPT)], cnt_out.at[pl.ds(r0, RPT)])


@functools.cache
def _make_spmm(with_count, cpt):
  out_acc = jax.ShapeDtypeStruct((NC, NACC, QW), jnp.float32)
  out_cnt = jax.ShapeDtypeStruct((NACC, CNTW), jnp.float32)
  hcpt = cpt // NPH
  scratch = [
      pltpu.VMEM((hcpt, CH), jnp.int32),
      pltpu.VMEM((hcpt, CH), jnp.int32),
      pltpu.VMEM((NBUF, CH, QW), jnp.float32),
  ]
  if with_count:
    scratch += [pltpu.VMEM((CH, CNTW), jnp.float32)]
  scratch += [
      pltpu.SemaphoreType.DMA((NBUF,)),
      pltpu.SemaphoreType.DMA((NBUF,)),
  ]
  if with_count:
    scratch += [pltpu.SemaphoreType.DMA]
  scratch += [
      pltpu.VMEM_SHARED((N_NODES, QW), jnp.float32),
      pltpu.VMEM_SHARED((NACC, QW), jnp.float32),
  ]
  if with_count:
    scratch += [pltpu.VMEM_SHARED((NACC, CNTW), jnp.float32)]
  return pl.kernel(
      functools.partial(_spmm_body, with_count, cpt),
      out_type=(out_acc, out_cnt) if with_count else out_acc,
      mesh=plsc.VectorSubcoreMesh(core_axis_name="c", subcore_axis_name="s"),
      scratch_types=scratch,
      compiler_params=pltpu.CompilerParams(use_tc_tiling_on_sc=False),
  )


def _dense_body(acca_ref, accb_ref, cnt_ref, x_ref, wl_ref, bl_ref, wr_ref,
                o_ref):
  cnt = cnt_ref[:, 0]
  inv = 1.0 / jnp.maximum(cnt, 1.0)
  y = bl_ref[...] + jnp.dot(x_ref[...], wr_ref[...],
                            preferred_element_type=jnp.float32)
  for j, acc_ref in ((0, acca_ref), (1, accb_ref)):
    for c2 in range(NC):
      part = acc_ref[c2] * inv[:, None]
      y = y + jnp.dot(part, wl_ref[pl.ds((2 * j + c2) * QW, QW), :],
                      preferred_element_type=jnp.float32)
  o_ref[...] = jnp.maximum(y, 0.0)


_DR = 1000


def _dense(acca, accb, cnt, x, wl, bl, wr):
  n = x.shape[0]
  grid = n // _DR
  return pl.pallas_call(
      _dense_body,
      grid=(grid,),
      in_specs=[
          pl.BlockSpec((NC, _DR, QW), lambda i: (0, i, 0)),
          pl.BlockSpec((NC, _DR, QW), lambda i: (0, i, 0)),
          pl.BlockSpec((_DR, CNTW), lambda i: (i, 0)),
          pl.BlockSpec((_DR, FDIM), lambda i: (i, 0)),
          pl.BlockSpec((FDIM, FDIM), lambda i: (0, 0)),
          pl.BlockSpec((1, FDIM), lambda i: (0, 0)),
          pl.BlockSpec((FDIM, FDIM), lambda i: (0, 0)),
      ],
      out_specs=pl.BlockSpec((_DR, FDIM), lambda i: (i, 0)),
      out_shape=jax.ShapeDtypeStruct((n, FDIM), jnp.float32),
  )(acca, accb, cnt, x, wl, bl, wr)


def kernel(x_human, x_bacterial, edge_index,
           h1_Wl, h1_bl, h1_Wr, h2_Wl, h2_bl, h2_Wr,
           b1_Wl, b1_bl, b1_Wr, b2_Wl, b2_bl, b2_Wr):
  src = edge_index[0]
  dst = edge_index[1]
  e = src.shape[0]
  cpt = -(-e // (NS * CH * NBUF * NPH)) * NBUF * NPH
  ep = NS * cpt * CH
  pad = ep - e
  gpad = jnp.zeros((pad,), jnp.int32)
  spad = jnp.full((pad,), DUMMY_ROW, jnp.int32)
  g_h = jnp.concatenate([dst, gpad]).reshape(NS, cpt, CH)
  s_h = jnp.concatenate([src, spad]).reshape(NS, cpt, CH)
  g_b = jnp.concatenate([src, gpad]).reshape(NS, cpt, CH)
  s_b = jnp.concatenate([dst, spad]).reshape(NS, cpt, CH)

  spmm_count = _make_spmm(True, cpt)
  spmm = _make_spmm(False, cpt)

  def qv(x):
    return x.reshape(-1, NQ, QW).transpose(1, 0, 2)

  qh = qv(x_human)
  qb = qv(x_bacterial)
  acc_h1a, cnt_h = spmm_count(qh[0:2], g_h, s_h)
  acc_h1b = spmm(qh[2:4], g_h, s_h)
  acc_b1a, cnt_b = spmm_count(qb[0:2], g_b, s_b)
  acc_b1b = spmm(qb[2:4], g_b, s_b)

  cnt_h = cnt_h[:N_NODES]
  cnt_b = cnt_b[:N_NODES]

  h1 = _dense(acc_h1a, acc_h1b, cnt_h, x_human,
              h1_Wl, h1_bl.reshape(1, -1), h1_Wr)
  b1 = _dense(acc_b1a, acc_b1b, cnt_b, x_bacterial,
              b1_Wl, b1_bl.reshape(1, -1), b1_Wr)

  qh1 = qv(h1)
  qb1 = qv(b1)
  acc_h2a = spmm(qh1[0:2], g_h, s_h)
  acc_h2b = spmm(qh1[2:4], g_h, s_h)
  acc_b2a = spmm(qb1[0:2], g_b, s_b)
  acc_b2b = spmm(qb1[2:4], g_b, s_b)

  h2 = _dense(acc_h2a, acc_h2b, cnt_h, h1, h2_Wl, h2_bl.reshape(1, -1), h2_Wr)
  b2 = _dense(acc_b2a, acc_b2b, cnt_b, b1, b2_Wl, b2_bl.reshape(1, -1), b2_Wr)
  return (h2, b2)


# speedup vs baseline: 2.0144x; 1.0714x over previous
import functools

import jax
import jax.numpy as jnp
from jax import lax
from jax.experimental import pallas as pl
from jax.experimental.pallas import tpu as pltpu
from jax.experimental.pallas import tpu_sc as plsc

N_NODES = 10000
FDIM = 128
QW = 32
NQ = FDIM // QW
NC = 2
NS = 16
CH = 128
CNTW = 16
NBUF = 8
NPH = 2

NACC = 10240
RPT = NACC // NS
NTT = N_NODES // NS
DUMMY_ROW = N_NODES


def _spmm_body(with_count, cpt, table, gidx, sidx, *refs):
  if with_count:
    (acc_out, cnt_out, gi_all, si_all, rows, ones_v,
     gsem, ssem, csem, tab_sh, acc_sh, cnt_sh) = refs
  else:
    acc_out, gi_all, si_all, rows, gsem, ssem, tab_sh, acc_sh = refs

  c = lax.axis_index("c")
  s = lax.axis_index("s")
  on_sc0 = c == 0
  zero16 = jnp.zeros((16,), jnp.float32)
  one16 = jnp.ones((16,), jnp.float32)
  hcpt = cpt // NPH
  ngroups = hcpt // NBUF

  t0 = s * NTT
  pltpu.sync_copy(table.at[c, pl.ds(t0, NTT)], tab_sh.at[pl.ds(t0, NTT)])

  def zrow(i, _):
    for k in range(QW // 16):
      rows[0, i, pl.ds(k * 16, 16)] = zero16
    return 0
  lax.fori_loop(0, CH, zrow, 0)
  r0 = pl.multiple_of(s * RPT, CH)
  for k in range(RPT // CH):
    pltpu.sync_copy(rows.at[0], acc_sh.at[pl.ds(r0 + k * CH, CH)])

  if with_count:
    @pl.when(on_sc0)
    def _():
      def zc(i, _):
        ones_v[i, :] = zero16
        return 0
      lax.fori_loop(0, CH, zc, 0)
      for k in range(RPT // CH):
        pltpu.sync_copy(ones_v.at[pl.ds(0, CH)],
                        cnt_sh.at[pl.ds(r0 + k * CH, CH)])

      def fo(i, _):
        ones_v[i, :] = one16
        return 0
      lax.fori_loop(0, CH, fo, 0)
  plsc.subcore_barrier()

  def gstart(chunk, b):
    pltpu.async_copy(tab_sh.at[gi_all.at[chunk]], rows.at[b], gsem.at[b])

  def gwait(b):
    pltpu.make_async_copy(tab_sh.at[gi_all.at[0]], rows.at[b],
                          gsem.at[b]).wait()

  def sstart(chunk, b):
    pltpu.async_copy(rows.at[b], acc_sh.at[si_all.at[chunk]], ssem.at[b],
                     add=True)

  def swait(b):
    pltpu.make_async_copy(rows.at[b], acc_sh.at[si_all.at[0]],
                          ssem.at[b]).wait()

  for ph in range(NPH):
    pltpu.sync_copy(gidx.at[s, pl.ds(ph * hcpt, hcpt)], gi_all)
    pltpu.sync_copy(sidx.at[s, pl.ds(ph * hcpt, hcpt)], si_all)

    for b in range(NBUF):
      gstart(b, b)

    def group(i, _):
      i0 = i * NBUF
      for b in range(NBUF):
        gwait(b)
        sstart(i0 + b, b)
        if with_count:
          @pl.when(on_sc0)
          def _():
            pltpu.async_copy(ones_v, cnt_sh.at[si_all.at[i0 + b]], csem,
                             add=True)
      for b in range(NBUF):
        swait(b)

        @pl.when(i < ngroups - 1)
        def _():
          gstart(i0 + NBUF + b, b)
      return 0
    lax.fori_loop(0, ngroups, group, 0)

    if with_count:
      @pl.when(on_sc0)
      def _():
        def drain(j, _):
          pltpu.make_async_copy(ones_v, cnt_sh.at[si_all.at[0]], csem).wait()
          return 0
        lax.fori_loop(0, hcpt, drain, 0)
  plsc.subcore_barrier()

  pltpu.sync_copy(acc_sh.at[pl.ds(r0, RPT)], acc_out.at[c, pl.ds(r0, RPT)])
  if with_count:
    @pl.when(on_sc0)
    def _():
      pltpu.sync_copy(cnt_sh.at[pl.ds(r0, RPT)], cnt_out.at[pl.ds(r0, RPT)])


@functools.cache
def _make_spmm(with_count, cpt):
  out_acc = jax.ShapeDtypeStruct((NC, NACC, QW), jnp.float32)
  out_cnt = jax.ShapeDtypeStruct((NACC, CNTW), jnp.float32)
  hcpt = cpt // NPH
  scratch = [
      pltpu.VMEM((hcpt, CH), jnp.int32),
      pltpu.VMEM((hcpt, CH), jnp.int32),
      pltpu.VMEM((NBUF, CH, QW), jnp.float32),
  ]
  if with_count:
    scratch += [pltpu.VMEM((CH, CNTW), jnp.float32)]
  scratch += [
      pltpu.SemaphoreType.DMA((NBUF,)),
      pltpu.SemaphoreType.DMA((NBUF,)),
  ]
  if with_count:
    scratch += [pltpu.SemaphoreType.DMA]
  scratch += [
      pltpu.VMEM_SHARED((N_NODES, QW), jnp.float32),
      pltpu.VMEM_SHARED((NACC, QW), jnp.float32),
  ]
  if with_count:
    scratch += [pltpu.VMEM_SHARED((NACC, CNTW), jnp.float32)]
  return pl.kernel(
      functools.partial(_spmm_body, with_count, cpt),
      out_type=(out_acc, out_cnt) if with_count else out_acc,
      mesh=plsc.VectorSubcoreMesh(core_axis_name="c", subcore_axis_name="s"),
      scratch_types=scratch,
      compiler_params=pltpu.CompilerParams(use_tc_tiling_on_sc=False),
  )


def _dense_body(acca_ref, accb_ref, cnt_ref, x_ref, wl_ref, bl_ref, wr_ref,
                o_ref):
  cnt = cnt_ref[:, 0]
  inv = 1.0 / jnp.maximum(cnt, 1.0)
  y = bl_ref[...] + jnp.dot(x_ref[...], wr_ref[...],
                            preferred_element_type=jnp.float32)
  for j, acc_ref in ((0, acca_ref), (1, accb_ref)):
    for c2 in range(NC):
      part = acc_ref[c2] * inv[:, None]
      y = y + jnp.dot(part, wl_ref[pl.ds((2 * j + c2) * QW, QW), :],
                      preferred_element_type=jnp.float32)
  o_ref[...] = jnp.maximum(y, 0.0)


_DR = 1000


def _dense(acca, accb, cnt, x, wl, bl, wr):
  n = x.shape[0]
  grid = n // _DR
  return pl.pallas_call(
      _dense_body,
      grid=(grid,),
      in_specs=[
          pl.BlockSpec((NC, _DR, QW), lambda i: (0, i, 0)),
          pl.BlockSpec((NC, _DR, QW), lambda i: (0, i, 0)),
          pl.BlockSpec((_DR, CNTW), lambda i: (i, 0)),
          pl.BlockSpec((_DR, FDIM), lambda i: (i, 0)),
          pl.BlockSpec((FDIM, FDIM), lambda i: (0, 0)),
          pl.BlockSpec((1, FDIM), lambda i: (0, 0)),
          pl.BlockSpec((FDIM, FDIM), lambda i: (0, 0)),
      ],
      out_specs=pl.BlockSpec((_DR, FDIM), lambda i: (i, 0)),
      out_shape=jax.ShapeDtypeStruct((n, FDIM), jnp.float32),
  )(acca, accb, cnt, x, wl, bl, wr)


def kernel(x_human, x_bacterial, edge_index,
           h1_Wl, h1_bl, h1_Wr, h2_Wl, h2_bl, h2_Wr,
           b1_Wl, b1_bl, b1_Wr, b2_Wl, b2_bl, b2_Wr):
  src = edge_index[0]
  dst = edge_index[1]
  e = src.shape[0]
  cpt = -(-e // (NS * CH * NBUF * NPH)) * NBUF * NPH
  ep = NS * cpt * CH
  pad = ep - e
  gpad = jnp.zeros((pad,), jnp.int32)
  spad = jnp.full((pad,), DUMMY_ROW, jnp.int32)
  g_h = jnp.concatenate([dst, gpad]).reshape(NS, cpt, CH)
  s_h = jnp.concatenate([src, spad]).reshape(NS, cpt, CH)
  g_b = jnp.concatenate([src, gpad]).reshape(NS, cpt, CH)
  s_b = jnp.concatenate([dst, spad]).reshape(NS, cpt, CH)

  spmm_count = _make_spmm(True, cpt)
  spmm = _make_spmm(False, cpt)

  def qv(x):
    return x.reshape(-1, NQ, QW).transpose(1, 0, 2)

  qh = qv(x_human)
  qb = qv(x_bacterial)
  acc_h1a, cnt_h = spmm_count(qh[0:2], g_h, s_h)
  acc_h1b = spmm(qh[2:4], g_h, s_h)
  acc_b1a, cnt_b = spmm_count(qb[0:2], g_b, s_b)
  acc_b1b = spmm(qb[2:4], g_b, s_b)

  cnt_h = cnt_h[:N_NODES]
  cnt_b = cnt_b[:N_NODES]

  h1 = _dense(acc_h1a, acc_h1b, cnt_h, x_human,
              h1_Wl, h1_bl.reshape(1, -1), h1_Wr)
  b1 = _dense(acc_b1a, acc_b1b, cnt_b, x_bacterial,
              b1_Wl, b1_bl.reshape(1, -1), b1_Wr)

  qh1 = qv(h1)
  qb1 = qv(b1)
  acc_h2a = spmm(qh1[0:2], g_h, s_h)
  acc_h2b = spmm(qh1[2:4], g_h, s_h)
  acc_b2a = spmm(qb1[0:2], g_b, s_b)
  acc_b2b = spmm(qb1[2:4], g_b, s_b)

  h2 = _dense(acc_h2a, acc_h2b, cnt_h, h1, h2_Wl, h2_bl.reshape(1, -1), h2_Wr)
  b2 = _dense(acc_b2a, acc_b2b, cnt_b, b1, b2_Wl, b2_bl.reshape(1, -1), b2_Wr)
  return (h2, b2)
